# baseline (speedup 1.0000x reference)
import functools

import jax
import jax.numpy as jnp
from jax import lax
from jax.experimental import pallas as pl
from jax.experimental.pallas import tpu as pltpu
from jax.experimental.pallas import tpu_sc as plsc

N = 10000
E = 320000
D_FEAT = 128
TYPE_DIM = 32
NUM_NODE_TYPES = 8
NUM_REL = 8
HEADS = 8
OUT = 16
HC = HEADS * OUT
NEG_SLOPE = 0.2

NC = 2
NS = 16
NW = NC * NS
EDGES_PER_W = E // NW
EB = 80
ITERS = EDGES_PER_W // EB
NPAD = 10240
ROWS_PER_SUB = NPAD // NS
DROWS = NPAD * HEADS // HC

ROW_BLK = 1000
GRID_N = N // ROW_BLK


def _prep_body(x_ref, ids_ref, emb_ref, w_ref, q_ref, k_ref,
               xr_ref, qn_ref, kn_ref):
    xb = x_ref[...]
    ids = ids_ref[...]
    onehot = (ids == lax.broadcasted_iota(jnp.int32, (1, NUM_NODE_TYPES), 1))
    onehot = onehot.astype(jnp.float32)
    temb = jnp.dot(onehot, emb_ref[...],
                   preferred_element_type=jnp.float32)
    zpad = jnp.zeros((HC, OUT - HEADS), jnp.float32)
    qp = jnp.concatenate([q_ref[...], zpad], axis=1)
    kp = jnp.concatenate([k_ref[...], zpad], axis=1)
    q_pieces = []
    k_pieces = []
    for r in range(NUM_REL):
        w1 = w_ref[r, :D_FEAT, :]
        w2 = w_ref[r, D_FEAT:, :]
        xr_r = (jnp.dot(xb, w1, preferred_element_type=jnp.float32)
                + jnp.dot(temb, w2, preferred_element_type=jnp.float32))
        xr_ref[:, r, :] = xr_r
        q_pieces.append(jnp.dot(xr_r, qp, preferred_element_type=jnp.float32))
        k_pieces.append(jnp.dot(xr_r, kp, preferred_element_type=jnp.float32))
    qn_ref[...] = jnp.concatenate(q_pieces, axis=1)
    kn_ref[...] = jnp.concatenate(k_pieces, axis=1)


def _prep_call(x, ids2, type_emb_table, weight, q, k):
    return pl.pallas_call(
        _prep_body,
        grid=(GRID_N,),
        in_specs=[
            pl.BlockSpec((ROW_BLK, D_FEAT), lambda i: (i, 0)),
            pl.BlockSpec((ROW_BLK, 1), lambda i: (i, 0)),
            pl.BlockSpec((NUM_NODE_TYPES, TYPE_DIM), lambda i: (0, 0)),
            pl.BlockSpec((NUM_REL, D_FEAT + TYPE_DIM, HC), lambda i: (0, 0, 0)),
            pl.BlockSpec((HC, HEADS), lambda i: (0, 0)),
            pl.BlockSpec((HC, HEADS), lambda i: (0, 0)),
        ],
        out_specs=[
            pl.BlockSpec((ROW_BLK, NUM_REL, HC), lambda i: (i, 0, 0)),
            pl.BlockSpec((ROW_BLK, HC), lambda i: (i, 0)),
            pl.BlockSpec((ROW_BLK, HC), lambda i: (i, 0)),
        ],
        out_shape=[
            jax.ShapeDtypeStruct((N, NUM_REL, HC), jnp.float32),
            jax.ShapeDtypeStruct((N, HC), jnp.float32),
            jax.ShapeDtypeStruct((N, HC), jnp.float32),
        ],
    )(x, ids2, type_emb_table, weight, q, k)


def _den_body(esrc_hbm, edst_hbm, et_hbm, qn_hbm, kn_hbm,
              ev_out, out_d, den_local,
              srci0, dsti0, relp0, dstp0, relc0, qrows0, krows0, evbuf0,
              srci1, dsti1, relp1, dstp1, relc1, qrows1, krows1, evbuf1,
              sem_i0, sem_q0, sem_k0, sem_e0,
              sem_i1, sem_q1, sem_k1, sem_e1):
    c = lax.axis_index("c")
    s = lax.axis_index("s")
    zero = jnp.zeros((16,), jnp.float32)
    wid = s * NC + c
    base = wid * EDGES_PER_W
    dmask = lax.iota(jnp.int32, 16) < HEADS

    S0 = (srci0, dsti0, relp0, dstp0, relc0, qrows0, krows0, evbuf0,
          sem_i0, sem_q0, sem_k0, sem_e0)
    S1 = (srci1, dsti1, relp1, dstp1, relc1, qrows1, krows1, evbuf1,
          sem_i1, sem_q1, sem_k1, sem_e1)

    def zden(i, carry):
        den_local[pl.ds(i * 16, 16)] = zero
        return carry

    lax.fori_loop(0, DROWS * HC // 16, zden, 0)

    def fire_idx(j, sl):
        off = base + j * EB
        pltpu.async_copy(esrc_hbm.at[pl.ds(off, EB)], sl[0], sl[8])
        pltpu.async_copy(edst_hbm.at[pl.ds(off, EB)], sl[1], sl[8])
        pltpu.async_copy(et_hbm.at[pl.ds(off, EB)], sl[2], sl[8])

    def wait_idx(sl):
        pltpu.make_async_copy(esrc_hbm.at[pl.ds(0, EB)], sl[0], sl[8]).wait()
        pltpu.make_async_copy(edst_hbm.at[pl.ds(0, EB)], sl[1], sl[8]).wait()
        pltpu.make_async_copy(et_hbm.at[pl.ds(0, EB)], sl[2], sl[8]).wait()

    def prep(sl):
        for v in range(EB // 16):
            vsl = pl.ds(v * 16, 16)
            sl[3][vsl] = sl[1][vsl]
            sl[4][vsl] = sl[2][vsl]

    def fire_gather(sl):
        pltpu.async_copy(qn_hbm.at[sl[1]], sl[5], sl[9])
        pltpu.async_copy(kn_hbm.at[sl[0]], sl[6], sl[10])

    def wait_gather(sl):
        pltpu.make_async_copy(qn_hbm.at[sl[1]], sl[5], sl[9]).wait()
        pltpu.make_async_copy(kn_hbm.at[sl[0]], sl[6], sl[10]).wait()

    def fire_ev(i, sl):
        pltpu.async_copy(
            sl[7], ev_out.at[pl.ds((base + i * EB) * OUT, EB * OUT)], sl[11])

    def wait_ev(sl):
        pltpu.make_async_copy(
            sl[7], ev_out.at[pl.ds(0, EB * OUT)], sl[11]).wait()

    def compute(sl):
        dstp, relc, qrows, krows, evbuf = sl[3], sl[4], sl[5], sl[6], sl[7]

        def e_body(e, carry2):
            r16 = relc[pl.ds(e, 16)][0] * OUT
            dv = dstp[pl.ds(e, 16)][0]
            av = qrows[e, pl.ds(r16, 16)] + krows[e, pl.ds(r16, 16)]
            av = jnp.maximum(av, av * NEG_SLOPE)
            ev = jnp.exp(av)
            evbuf[pl.ds(e * OUT, 16)] = ev
            dslice = pl.ds(dv * HEADS, 16)
            den_local[dslice] = den_local[dslice] + jnp.where(dmask, ev, 0.0)
            return carry2

        lax.fori_loop(0, EB, e_body, 0, unroll=2)

    def step(i, cur, nxt):
        wait_idx(nxt)
        prep(nxt)
        fire_gather(nxt)
        wait_gather(cur)

        @pl.when(i <= ITERS - 3)
        def _():
            fire_idx(i + 2, cur)

        @pl.when(i >= 2)
        def _():
            wait_ev(cur)

        compute(cur)
        fire_ev(i, cur)

    fire_idx(0, S0)
    wait_idx(S0)
    prep(S0)
    fire_gather(S0)
    fire_idx(1, S1)

    def pair(g, carry):
        step(2 * g, S0, S1)
        step(2 * g + 1, S1, S0)
        return carry

    lax.fori_loop(0, (ITERS - 1) // 2, pair, 0)

    wait_gather(S0)
    wait_ev(S0)
    compute(S0)
    fire_ev(ITERS - 1, S0)
    wait_ev(S1)
    wait_ev(S0)
    pltpu.sync_copy(den_local, out_d.at[wid])


_den_call = functools.partial(
    pl.kernel,
    out_type=[
        jax.ShapeDtypeStruct((E * OUT,), jnp.float32),
        jax.ShapeDtypeStruct((NW, DROWS * HC), jnp.float32),
    ],
    mesh=plsc.VectorSubcoreMesh(core_axis_name="c", subcore_axis_name="s"),
    scratch_types=[
        pltpu.VMEM((DROWS * HC,), jnp.float32),
    ] + 2 * [
        pltpu.VMEM((EB,), jnp.int32),
        pltpu.VMEM((EB,), jnp.int32),
        pltpu.VMEM((EB,), jnp.int32),
        pltpu.VMEM((EB + 16,), jnp.int32),
        pltpu.VMEM((EB + 16,), jnp.int32),
        pltpu.VMEM((EB, HC), jnp.float32),
        pltpu.VMEM((EB, HC), jnp.float32),
        pltpu.VMEM((EB * OUT,), jnp.float32),
    ] + 8 * [pltpu.SemaphoreType.DMA],
)(_den_body)


def _msg_body(esrc_hbm, edst_hbm, et_hbm, xr_hbm, ev_hbm,
              out_m, acc,
              srci0, dsti0, relp0, fsrc0, dsts0, xrows0, evbuf0, msg0,
              srci1, dsti1, relp1, fsrc1, dsts1, xrows1, evbuf1, msg1,
              sem_i0, sem_x0, sem_e0, sem_s0,
              sem_i1, sem_x1, sem_e1, sem_s1):
    c = lax.axis_index("c")
    s = lax.axis_index("s")
    zero = jnp.zeros((16,), jnp.float32)
    wid = s * NC + c
    base = wid * EDGES_PER_W

    S0 = (srci0, dsti0, relp0, fsrc0, dsts0, xrows0, evbuf0, msg0,
          sem_i0, sem_x0, sem_e0, sem_s0)
    S1 = (srci1, dsti1, relp1, fsrc1, dsts1, xrows1, evbuf1, msg1,
          sem_i1, sem_x1, sem_e1, sem_s1)

    def zrow_m(i, carry):
        for j in range(HC // 16):
            msg0[i, pl.ds(j * 16, 16)] = zero
        return carry

    lax.fori_loop(0, EB, zrow_m, 0)
    for t in range(ROWS_PER_SUB // EB):
        pltpu.sync_copy(msg0, acc.at[pl.ds(s * ROWS_PER_SUB + t * EB, EB)])
    plsc.subcore_barrier()

    def fire_idx(j, sl):
        off = base + j * EB
        pltpu.async_copy(esrc_hbm.at[pl.ds(off, EB)], sl[0], sl[8])
        pltpu.async_copy(edst_hbm.at[pl.ds(off, EB)], sl[1], sl[8])
        pltpu.async_copy(et_hbm.at[pl.ds(off, EB)], sl[2], sl[8])

    def wait_idx(sl):
        pltpu.make_async_copy(esrc_hbm.at[pl.ds(0, EB)], sl[0], sl[8]).wait()
        pltpu.make_async_copy(edst_hbm.at[pl.ds(0, EB)], sl[1], sl[8]).wait()
        pltpu.make_async_copy(et_hbm.at[pl.ds(0, EB)], sl[2], sl[8]).wait()

    def prep(sl):
        for v in range(EB // 16):
            vsl = pl.ds(v * 16, 16)
            sl[3][vsl] = sl[0][vsl] * NUM_REL + sl[2][vsl]

    def fire_x(sl):
        pltpu.async_copy(xr_hbm.at[sl[3]], sl[5], sl[9])

    def wait_x(sl):
        pltpu.make_async_copy(xr_hbm.at[sl[3]], sl[5], sl[9]).wait()

    def fire_evload(j, sl):
        off = (base + j * EB) * OUT
        pltpu.async_copy(ev_hbm.at[pl.ds(off, EB * OUT)], sl[6], sl[10])

    def wait_evload(sl):
        pltpu.make_async_copy(
            ev_hbm.at[pl.ds(0, EB * OUT)], sl[6], sl[10]).wait()

    def stash_dsts(sl):
        for v in range(EB // 16):
            vsl = pl.ds(v * 16, 16)
            sl[4][vsl] = sl[1][vsl]

    def fire_scatter(sl):
        pltpu.async_copy(sl[7], acc.at[sl[4]], sl[11], add=True)

    def wait_scatter(sl):
        pltpu.make_async_copy(sl[7], acc.at[sl[4]], sl[11]).wait()

    def compute(sl):
        xrows, evbuf, msg = sl[5], sl[6], sl[7]

        def e_body(e, carry2):
            ev = evbuf[pl.ds(e * OUT, 16)]
            for h in range(HEADS):
                sc = ev[h]
                msg[e, pl.ds(h * OUT, 16)] = xrows[e, pl.ds(h * OUT, 16)] * sc
            return carry2

        lax.fori_loop(0, EB, e_body, 0, unroll=2)

    def step(i, cur, nxt):
        wait_idx(nxt)
        prep(nxt)
        fire_x(nxt)
        wait_x(cur)

        @pl.when(i >= 2)
        def _():
            wait_scatter(cur)

        stash_dsts(cur)

        @pl.when(i <= ITERS - 3)
        def _():
            fire_idx(i + 2, cur)

        wait_evload(cur)
        compute(cur)
        fire_scatter(cur)

        @pl.when(i <= ITERS - 3)
        def _():
            fire_evload(i + 2, cur)

    fire_idx(0, S0)
    wait_idx(S0)
    prep(S0)
    fire_x(S0)
    fire_idx(1, S1)
    fire_evload(0, S0)
    fire_evload(1, S1)

    def pair(g, carry):
        step(2 * g, S0, S1)
        step(2 * g + 1, S1, S0)
        return carry

    lax.fori_loop(0, (ITERS - 1) // 2, pair, 0)

    wait_x(S0)
    wait_scatter(S0)
    stash_dsts(S0)
    wait_evload(S0)
    compute(S0)
    fire_scatter(S0)
    wait_scatter(S1)
    wait_scatter(S0)

    plsc.subcore_barrier()
    pltpu.sync_copy(acc.at[pl.ds(s * ROWS_PER_SUB, ROWS_PER_SUB)],
                    out_m.at[c, pl.ds(s * ROWS_PER_SUB, ROWS_PER_SUB)])


_msg_call = functools.partial(
    pl.kernel,
    out_type=jax.ShapeDtypeStruct((NC, NPAD, HC), jnp.float32),
    mesh=plsc.VectorSubcoreMesh(core_axis_name="c", subcore_axis_name="s"),
    scratch_types=[
        pltpu.VMEM_SHARED((NPAD, HC), jnp.float32),
    ] + 2 * [
        pltpu.VMEM((EB,), jnp.int32),
        pltpu.VMEM((EB,), jnp.int32),
        pltpu.VMEM((EB,), jnp.int32),
        pltpu.VMEM((EB,), jnp.int32),
        pltpu.VMEM((EB,), jnp.int32),
        pltpu.VMEM((EB, HC), jnp.float32),
        pltpu.VMEM((EB * OUT,), jnp.float32),
        pltpu.VMEM((EB, HC), jnp.float32),
    ] + 8 * [pltpu.SemaphoreType.DMA],
)(_msg_body)


def _dsum_body(pd_ref, out_ref):
    out_ref[...] = jnp.sum(pd_ref[...], axis=0)


def _dsum_call(part_d):
    blk = DROWS // 5
    return pl.pallas_call(
        _dsum_body,
        grid=(5,),
        in_specs=[pl.BlockSpec((NW, blk, HC), lambda i: (0, i, 0))],
        out_specs=pl.BlockSpec((blk, HC), lambda i: (i, 0)),
        out_shape=jax.ShapeDtypeStruct((DROWS, HC), jnp.float32),
    )(part_d)


def _final_body(pm_ref, dex_ref, bias_ref, out_ref):
    m = pm_ref[0] + pm_ref[1]
    out = m / (dex_ref[...] + 1e-16) + bias_ref[...]
    out_ref[...] = jnp.maximum(out, 0.0)


def _final_call(part_m, dex, bias2):
    return pl.pallas_call(
        _final_body,
        grid=(GRID_N,),
        in_specs=[
            pl.BlockSpec((NC, ROW_BLK, HC), lambda i: (0, i, 0)),
            pl.BlockSpec((ROW_BLK, HC), lambda i: (i, 0)),
            pl.BlockSpec((1, HC), lambda i: (0, 0)),
        ],
        out_specs=pl.BlockSpec((ROW_BLK, HC), lambda i: (i, 0)),
        out_shape=jax.ShapeDtypeStruct((N, HC), jnp.float32),
    )(part_m, dex, bias2)


def kernel(x, node_type_ids, edge_index, edge_type, type_emb_table, weight,
           q, k, bias):
    ids2 = node_type_ids.astype(jnp.int32).reshape(N, 1)
    xr3, qn2, kn2 = _prep_call(x, ids2, type_emb_table, weight, q, k)
    xr_flat = xr3.reshape(N * NUM_REL, HC)
    ei = edge_index.astype(jnp.int32)
    esrc, edst = ei[0], ei[1]
    et = edge_type.astype(jnp.int32)
    evtmp, part_d = _den_call(esrc, edst, et, qn2, kn2)
    part_m = _msg_call(esrc, edst, et, xr_flat, evtmp)
    dsum = _dsum_call(part_d.reshape(NW, DROWS, HC))
    dex = jnp.broadcast_to(
        dsum.reshape(NPAD, HEADS, 1), (NPAD, HEADS, OUT)).reshape(NPAD, HC)
    return _final_call(part_m, dex, bias.reshape(1, HC))


# speedup vs baseline: 1.4134x; 1.4134x over previous
import functools

import jax
import jax.numpy as jnp
from jax import lax
from jax.experimental import pallas as pl
from jax.experimental.pallas import tpu as pltpu
from jax.experimental.pallas import tpu_sc as plsc

N = 10000
E = 320000
D_FEAT = 128
TYPE_DIM = 32
NUM_NODE_TYPES = 8
NUM_REL = 8
HEADS = 8
OUT = 16
HC = HEADS * OUT
NEG_SLOPE = 0.2

NC = 2
NS = 16
NW = NC * NS
EDGES_PER_W = E // NW
EB = 80
ITERS = EDGES_PER_W // EB
NPAD = 10240
ROWS_PER_SUB = NPAD // NS
DROWS = NPAD * HEADS // HC

ROW_BLK = 1000
GRID_N = N // ROW_BLK


def _prep_body(x_ref, ids_ref, emb_ref, w_ref, q_ref, k_ref,
               xr_ref, qn_ref, kn_ref):
    xb = x_ref[...]
    ids = ids_ref[...]
    onehot = (ids == lax.broadcasted_iota(jnp.int32, (1, NUM_NODE_TYPES), 1))
    onehot = onehot.astype(jnp.float32)
    temb = jnp.dot(onehot, emb_ref[...],
                   preferred_element_type=jnp.float32)
    zpad = jnp.zeros((HC, OUT - HEADS), jnp.float32)
    qp = jnp.concatenate([q_ref[...], zpad], axis=1)
    kp = jnp.concatenate([k_ref[...], zpad], axis=1)
    q_pieces = []
    k_pieces = []
    for r in range(NUM_REL):
        w1 = w_ref[r, :D_FEAT, :]
        w2 = w_ref[r, D_FEAT:, :]
        xr_r = (jnp.dot(xb, w1, preferred_element_type=jnp.float32)
                + jnp.dot(temb, w2, preferred_element_type=jnp.float32))
        xr_ref[:, r, :] = xr_r
        q_pieces.append(jnp.dot(xr_r, qp, preferred_element_type=jnp.float32))
        k_pieces.append(jnp.dot(xr_r, kp, preferred_element_type=jnp.float32))
    qn_ref[...] = jnp.concatenate(q_pieces, axis=1)
    kn_ref[...] = jnp.concatenate(k_pieces, axis=1)


def _prep_call(x, ids2, type_emb_table, weight, q, k):
    return pl.pallas_call(
        _prep_body,
        grid=(GRID_N,),
        in_specs=[
            pl.BlockSpec((ROW_BLK, D_FEAT), lambda i: (i, 0)),
            pl.BlockSpec((ROW_BLK, 1), lambda i: (i, 0)),
            pl.BlockSpec((NUM_NODE_TYPES, TYPE_DIM), lambda i: (0, 0)),
            pl.BlockSpec((NUM_REL, D_FEAT + TYPE_DIM, HC), lambda i: (0, 0, 0)),
            pl.BlockSpec((HC, HEADS), lambda i: (0, 0)),
            pl.BlockSpec((HC, HEADS), lambda i: (0, 0)),
        ],
        out_specs=[
            pl.BlockSpec((ROW_BLK, NUM_REL, HC), lambda i: (i, 0, 0)),
            pl.BlockSpec((ROW_BLK, HC), lambda i: (i, 0)),
            pl.BlockSpec((ROW_BLK, HC), lambda i: (i, 0)),
        ],
        out_shape=[
            jax.ShapeDtypeStruct((N, NUM_REL, HC), jnp.float32),
            jax.ShapeDtypeStruct((N, HC), jnp.float32),
            jax.ShapeDtypeStruct((N, HC), jnp.float32),
        ],
    )(x, ids2, type_emb_table, weight, q, k)


def _den_body(esrc_hbm, edst_hbm, et_hbm, qn_hbm, kn_hbm,
              ev_out, out_d, den_local,
              srci0, dsti0, relp0, dstp0, relc0, qrows0, krows0, evbuf0,
              srci1, dsti1, relp1, dstp1, relc1, qrows1, krows1, evbuf1,
              sem_i0, sem_q0, sem_k0, sem_e0,
              sem_i1, sem_q1, sem_k1, sem_e1):
    c = lax.axis_index("c")
    s = lax.axis_index("s")
    zero = jnp.zeros((16,), jnp.float32)
    wid = s * NC + c
    base = wid * EDGES_PER_W
    dmask = lax.iota(jnp.int32, 16) < HEADS

    S0 = (srci0, dsti0, relp0, dstp0, relc0, qrows0, krows0, evbuf0,
          sem_i0, sem_q0, sem_k0, sem_e0)
    S1 = (srci1, dsti1, relp1, dstp1, relc1, qrows1, krows1, evbuf1,
          sem_i1, sem_q1, sem_k1, sem_e1)

    def zden(i, carry):
        den_local[pl.ds(i * 16, 16)] = zero
        return carry

    lax.fori_loop(0, DROWS * HC // 16, zden, 0)

    def fire_idx(j, sl):
        off = base + j * EB
        pltpu.async_copy(esrc_hbm.at[pl.ds(off, EB)], sl[0], sl[8])
        pltpu.async_copy(edst_hbm.at[pl.ds(off, EB)], sl[1], sl[8])
        pltpu.async_copy(et_hbm.at[pl.ds(off, EB)], sl[2], sl[8])

    def wait_idx(sl):
        pltpu.make_async_copy(esrc_hbm.at[pl.ds(0, EB)], sl[0], sl[8]).wait()
        pltpu.make_async_copy(edst_hbm.at[pl.ds(0, EB)], sl[1], sl[8]).wait()
        pltpu.make_async_copy(et_hbm.at[pl.ds(0, EB)], sl[2], sl[8]).wait()

    def prep(sl):
        for v in range(EB // 16):
            vsl = pl.ds(v * 16, 16)
            sl[3][vsl] = sl[1][vsl]
            sl[4][vsl] = sl[2][vsl]

    def fire_gather(sl):
        pltpu.async_copy(qn_hbm.at[sl[1]], sl[5], sl[9])
        pltpu.async_copy(kn_hbm.at[sl[0]], sl[6], sl[10])

    def wait_gather(sl):
        pltpu.make_async_copy(qn_hbm.at[sl[1]], sl[5], sl[9]).wait()
        pltpu.make_async_copy(kn_hbm.at[sl[0]], sl[6], sl[10]).wait()

    def fire_ev(i, sl):
        pltpu.async_copy(
            sl[7], ev_out.at[pl.ds((base + i * EB) * OUT, EB * OUT)], sl[11])

    def wait_ev(sl):
        pltpu.make_async_copy(
            sl[7], ev_out.at[pl.ds(0, EB * OUT)], sl[11]).wait()

    def compute(sl):
        dstp, relc, qrows, krows, evbuf = sl[3], sl[4], sl[5], sl[6], sl[7]

        def e_body(e, carry2):
            r16 = relc[pl.ds(e, 16)][0] * OUT
            dv = dstp[pl.ds(e, 16)][0]
            av = qrows[e, pl.ds(r16, 16)] + krows[e, pl.ds(r16, 16)]
            av = jnp.maximum(av, av * NEG_SLOPE)
            ev = jnp.exp(av)
            evbuf[pl.ds(e * OUT, 16)] = ev
            dslice = pl.ds(dv * HEADS, 16)
            den_local[dslice] = den_local[dslice] + jnp.where(dmask, ev, 0.0)
            return carry2

        lax.fori_loop(0, EB, e_body, 0, unroll=2)

    def step(i, cur, nxt):
        wait_idx(nxt)
        prep(nxt)
        fire_gather(nxt)
        wait_gather(cur)

        @pl.when(i <= ITERS - 3)
        def _():
            fire_idx(i + 2, cur)

        @pl.when(i >= 2)
        def _():
            wait_ev(cur)

        compute(cur)
        fire_ev(i, cur)

    fire_idx(0, S0)
    wait_idx(S0)
    prep(S0)
    fire_gather(S0)
    fire_idx(1, S1)

    def pair(g, carry):
        step(2 * g, S0, S1)
        step(2 * g + 1, S1, S0)
        return carry

    lax.fori_loop(0, (ITERS - 1) // 2, pair, 0)

    wait_gather(S0)
    wait_ev(S0)
    compute(S0)
    fire_ev(ITERS - 1, S0)
    wait_ev(S1)
    wait_ev(S0)
    pltpu.sync_copy(den_local, out_d.at[wid])


_den_call = functools.partial(
    pl.kernel,
    out_type=[
        jax.ShapeDtypeStruct((E * OUT,), jnp.float32),
        jax.ShapeDtypeStruct((NW, DROWS * HC), jnp.float32),
    ],
    mesh=plsc.VectorSubcoreMesh(core_axis_name="c", subcore_axis_name="s"),
    scratch_types=[
        pltpu.VMEM((DROWS * HC,), jnp.float32),
    ] + 2 * [
        pltpu.VMEM((EB,), jnp.int32),
        pltpu.VMEM((EB,), jnp.int32),
        pltpu.VMEM((EB,), jnp.int32),
        pltpu.VMEM((EB + 16,), jnp.int32),
        pltpu.VMEM((EB + 16,), jnp.int32),
        pltpu.VMEM((EB, HC), jnp.float32),
        pltpu.VMEM((EB, HC), jnp.float32),
        pltpu.VMEM((EB * OUT,), jnp.float32),
    ] + 8 * [pltpu.SemaphoreType.DMA],
)(_den_body)


def _msg_body(esrc_hbm, edst_hbm, et_hbm, xr_hbm, ev_hbm,
              out_m, acc,
              srci0, dsti0, relp0, fsrc0, dsts0, xrows0, evbuf0, msg0,
              srci1, dsti1, relp1, fsrc1, dsts1, xrows1, evbuf1, msg1,
              sem_i0, sem_x0, sem_e0, sem_s0,
              sem_i1, sem_x1, sem_e1, sem_s1):
    c = lax.axis_index("c")
    s = lax.axis_index("s")
    zero = jnp.zeros((16,), jnp.float32)
    wid = s * NC + c
    base = wid * EDGES_PER_W

    S0 = (srci0, dsti0, relp0, fsrc0, dsts0, xrows0, evbuf0, msg0,
          sem_i0, sem_x0, sem_e0, sem_s0)
    S1 = (srci1, dsti1, relp1, fsrc1, dsts1, xrows1, evbuf1, msg1,
          sem_i1, sem_x1, sem_e1, sem_s1)

    def zrow_m(i, carry):
        for j in range(HC // 16):
            msg0[i, pl.ds(j * 16, 16)] = zero
        return carry

    lax.fori_loop(0, EB, zrow_m, 0)
    for t in range(ROWS_PER_SUB // EB):
        pltpu.sync_copy(msg0, acc.at[pl.ds(s * ROWS_PER_SUB + t * EB, EB)])
    plsc.subcore_barrier()

    def fire_idx(j, sl):
        off = base + j * EB
        pltpu.async_copy(esrc_hbm.at[pl.ds(off, EB)], sl[0], sl[8])
        pltpu.async_copy(edst_hbm.at[pl.ds(off, EB)], sl[1], sl[8])
        pltpu.async_copy(et_hbm.at[pl.ds(off, EB)], sl[2], sl[8])

    def wait_idx(sl):
        pltpu.make_async_copy(esrc_hbm.at[pl.ds(0, EB)], sl[0], sl[8]).wait()
        pltpu.make_async_copy(edst_hbm.at[pl.ds(0, EB)], sl[1], sl[8]).wait()
        pltpu.make_async_copy(et_hbm.at[pl.ds(0, EB)], sl[2], sl[8]).wait()

    def prep(sl):
        for v in range(EB // 16):
            vsl = pl.ds(v * 16, 16)
            sl[3][vsl] = sl[0][vsl] * NUM_REL + sl[2][vsl]

    def fire_x(sl):
        pltpu.async_copy(xr_hbm.at[sl[3]], sl[5], sl[9])

    def wait_x(sl):
        pltpu.make_async_copy(xr_hbm.at[sl[3]], sl[5], sl[9]).wait()

    def fire_evload(j, sl):
        off = (base + j * EB) * OUT
        pltpu.async_copy(ev_hbm.at[pl.ds(off, EB * OUT)], sl[6], sl[10])

    def wait_evload(sl):
        pltpu.make_async_copy(
            ev_hbm.at[pl.ds(0, EB * OUT)], sl[6], sl[10]).wait()

    def stash_dsts(sl):
        for v in range(EB // 16):
            vsl = pl.ds(v * 16, 16)
            sl[4][vsl] = sl[1][vsl]

    def fire_scatter(sl):
        pltpu.async_copy(sl[7], acc.at[sl[4]], sl[11], add=True)

    def wait_scatter(sl):
        pltpu.make_async_copy(sl[7], acc.at[sl[4]], sl[11]).wait()

    def compute(sl):
        xrows, evbuf, msg = sl[5], sl[6], sl[7]

        def e_body(e, carry2):
            ev = evbuf[pl.ds(e * OUT, 16)]
            for h in range(HEADS):
                sc = ev[h]
                msg[e, pl.ds(h * OUT, 16)] = xrows[e, pl.ds(h * OUT, 16)] * sc
            return carry2

        lax.fori_loop(0, EB, e_body, 0)

    def step(i, cur, nxt):
        wait_idx(nxt)
        prep(nxt)
        fire_x(nxt)
        wait_x(cur)

        @pl.when(i >= 2)
        def _():
            wait_scatter(cur)

        stash_dsts(cur)

        @pl.when(i <= ITERS - 3)
        def _():
            fire_idx(i + 2, cur)

        wait_evload(cur)
        compute(cur)
        fire_scatter(cur)

        @pl.when(i <= ITERS - 3)
        def _():
            fire_evload(i + 2, cur)

    fire_idx(0, S0)
    wait_idx(S0)
    prep(S0)
    fire_x(S0)
    fire_idx(1, S1)
    fire_evload(0, S0)
    fire_evload(1, S1)

    def pair(g, carry):
        step(2 * g, S0, S1)
        step(2 * g + 1, S1, S0)
        return carry

    lax.fori_loop(0, (ITERS - 1) // 2, pair, 0)

    wait_x(S0)
    wait_scatter(S0)
    stash_dsts(S0)
    wait_evload(S0)
    compute(S0)
    fire_scatter(S0)
    wait_scatter(S1)
    wait_scatter(S0)

    plsc.subcore_barrier()
    pltpu.sync_copy(acc.at[pl.ds(s * ROWS_PER_SUB, ROWS_PER_SUB)],
                    out_m.at[c, pl.ds(s * ROWS_PER_SUB, ROWS_PER_SUB)])


_msg_call = functools.partial(
    pl.kernel,
    out_type=jax.ShapeDtypeStruct((NC, NPAD, HC), jnp.float32),
    mesh=plsc.VectorSubcoreMesh(core_axis_name="c", subcore_axis_name="s"),
    scratch_types=[
        pltpu.VMEM_SHARED((NPAD, HC), jnp.float32),
    ] + 2 * [
        pltpu.VMEM((EB,), jnp.int32),
        pltpu.VMEM((EB,), jnp.int32),
        pltpu.VMEM((EB,), jnp.int32),
        pltpu.VMEM((EB,), jnp.int32),
        pltpu.VMEM((EB,), jnp.int32),
        pltpu.VMEM((EB, HC), jnp.float32),
        pltpu.VMEM((EB * OUT,), jnp.float32),
        pltpu.VMEM((EB, HC), jnp.float32),
    ] + 8 * [pltpu.SemaphoreType.DMA],
)(_msg_body)


def _dsum_body(pd_ref, out_ref):
    out_ref[...] = jnp.sum(pd_ref[...], axis=0)


def _dsum_call(part_d):
    blk = DROWS // 5
    return pl.pallas_call(
        _dsum_body,
        grid=(5,),
        in_specs=[pl.BlockSpec((NW, blk, HC), lambda i: (0, i, 0))],
        out_specs=pl.BlockSpec((blk, HC), lambda i: (i, 0)),
        out_shape=jax.ShapeDtypeStruct((DROWS, HC), jnp.float32),
    )(part_d)


def _final_body(pm_ref, dex_ref, bias_ref, out_ref):
    m = pm_ref[0] + pm_ref[1]
    out = m / (dex_ref[...] + 1e-16) + bias_ref[...]
    out_ref[...] = jnp.maximum(out, 0.0)


def _final_call(part_m, dex, bias2):
    return pl.pallas_call(
        _final_body,
        grid=(GRID_N,),
        in_specs=[
            pl.BlockSpec((NC, ROW_BLK, HC), lambda i: (0, i, 0)),
            pl.BlockSpec((ROW_BLK, HC), lambda i: (i, 0)),
            pl.BlockSpec((1, HC), lambda i: (0, 0)),
        ],
        out_specs=pl.BlockSpec((ROW_BLK, HC), lambda i: (i, 0)),
        out_shape=jax.ShapeDtypeStruct((N, HC), jnp.float32),
    )(part_m, dex, bias2)


def kernel(x, node_type_ids, edge_index, edge_type, type_emb_table, weight,
           q, k, bias):
    ids2 = node_type_ids.astype(jnp.int32).reshape(N, 1)
    xr3, qn2, kn2 = _prep_call(x, ids2, type_emb_table, weight, q, k)
    xr_flat = xr3.reshape(N * NUM_REL, HC)
    ei = edge_index.astype(jnp.int32)
    esrc, edst = ei[0], ei[1]
    et = edge_type.astype(jnp.int32)
    evtmp, part_d = _den_call(esrc, edst, et, qn2, kn2)
    part_m = _msg_call(esrc, edst, et, xr_flat, evtmp)
    dsum = _dsum_call(part_d.reshape(NW, DROWS, HC))
    dex = jnp.broadcast_to(
        dsum.reshape(NPAD, HEADS, 1), (NPAD, HEADS, OUT)).reshape(NPAD, HC)
    return _final_call(part_m, dex, bias.reshape(1, HC))
